# Initial kernel scaffold; baseline (speedup 1.0000x reference)
import jax
import jax.numpy as jnp
from jax.experimental import pallas as pl


def kernel(features, l_u_indices, l_d_indices, p_indices, p_values, W_p, b_p, W_d, b_d, a_src_d, a_dst_d):
    raise NotImplementedError("write your pallas kernel here")


# speedup vs baseline: 6.6977x; 6.6977x over previous
import functools

import jax
import jax.numpy as jnp
from jax import lax
from jax.experimental import pallas as pl
from jax.experimental.pallas import tpu as pltpu
from jax.experimental.pallas import tpu_sc as plsc

N = 10000
D = 128
E = 320000
NT = 16
NSC = 2
NGC = 8
GAT_PER_TILE = E // NT
NG_GAT = 160
CH_GAT = NG_GAT // NGC
P_PER_TILE = E // (NT * NSC)
NG_P = 80
CH_P = NG_P // NGC
PAD_DST = N
T_PAD = N + 16
ACC_ROWS = 10112
DEN_PER_TILE = 640
DEN_WORDS = NT * DEN_PER_TILE
ROWS_PER_TILE = ACC_ROWS // NT


def _dense_body(x_ref, wd_ref, bd_ref, wp_ref, bp_ref, a_ref,
                h_ref, hp_ref, st_ref, sm_ref, smem_ref):
    i = pl.program_id(0)
    x = x_ref[...]
    h = jnp.dot(x, wd_ref[...], preferred_element_type=jnp.float32) + bd_ref[...]
    h_ref[...] = h
    hp_ref[...] = jnp.dot(x, wp_ref[...], preferred_element_type=jnp.float32) + bp_ref[...]
    st = jnp.dot(h, a_ref[...], preferred_element_type=jnp.float32)
    st_ref[...] = st

    @pl.when(i == 0)
    def _():
        smem_ref[0] = jnp.float32(-3.0e38)

    blk_max = jnp.max(st[:, 0])
    smem_ref[0] = jnp.maximum(smem_ref[0], blk_max)

    @pl.when(i == pl.num_programs(0) - 1)
    def _():
        sm_ref[...] = jnp.full((8, 128), smem_ref[0], jnp.float32)


def _dense(x, W_d, b_d, W_p, b_p, A):
    blk = 1000
    grid = N // blk
    return pl.pallas_call(
        _dense_body,
        grid=(grid,),
        in_specs=[
            pl.BlockSpec((blk, D), lambda i: (i, 0)),
            pl.BlockSpec((D, D), lambda i: (0, 0)),
            pl.BlockSpec((1, D), lambda i: (0, 0)),
            pl.BlockSpec((D, D), lambda i: (0, 0)),
            pl.BlockSpec((1, D), lambda i: (0, 0)),
            pl.BlockSpec((D, D), lambda i: (0, 0)),
        ],
        out_specs=[
            pl.BlockSpec((blk, D), lambda i: (i, 0)),
            pl.BlockSpec((blk, D), lambda i: (i, 0)),
            pl.BlockSpec((blk, D), lambda i: (i, 0)),
            pl.BlockSpec((8, 128), lambda i: (0, 0)),
        ],
        out_shape=[
            jax.ShapeDtypeStruct((N, D), jnp.float32),
            jax.ShapeDtypeStruct((N, D), jnp.float32),
            jax.ShapeDtypeStruct((N, D), jnp.float32),
            jax.ShapeDtypeStruct((8, 128), jnp.float32),
        ],
        scratch_shapes=[pltpu.SMEM((1,), jnp.float32)],
    )(x, W_d, b_d, W_p, b_p, A)


_mesh = plsc.VectorSubcoreMesh(core_axis_name="c", subcore_axis_name="s",
                               num_cores=NSC, num_subcores=NT)


@functools.partial(
    pl.kernel,
    out_type=jax.ShapeDtypeStruct((NSC, ACC_ROWS, D), jnp.float32),
    mesh=_mesh,
    compiler_params=pltpu.CompilerParams(needs_layout_passes=False),
    scratch_types=[
        pltpu.VMEM((NGC, 128), jnp.int32),
        pltpu.VMEM((NGC, 128), jnp.int32),
        pltpu.VMEM((NGC, 128), jnp.float32),
        pltpu.VMEM((NGC, 128), jnp.float32),
        pltpu.VMEM((NGC, 128), jnp.float32),
        pltpu.VMEM((NGC, 128), jnp.float32),
        pltpu.VMEM((2, 128, D), jnp.float32),
        pltpu.VMEM((DEN_PER_TILE,), jnp.float32),
        pltpu.VMEM((16,), jnp.float32),
        pltpu.VMEM_SHARED((ACC_ROWS, D), jnp.float32),
        pltpu.VMEM_SHARED((DEN_WORDS,), jnp.float32),
        pltpu.SemaphoreType.DMA,
        pltpu.SemaphoreType.DMA,
        pltpu.SemaphoreType.DMA,
    ],
)
def _sc_aggregate(gsrc, gdst, psrc, pdst, pvals, h_hbm, hp_hbm, s_hbm, t_hbm,
                  sm_hbm, out_hbm, srcb, dstb, wb, sbuf, tbuf, denb,
                  rowb, zb, smv, acc_sh, den_sh, semA, semR0, semR1):
    c = lax.axis_index("c")
    sid = lax.axis_index("s")
    wid = c * NT + sid
    semR = (semR0, semR1)
    neg = jnp.float32(0.2)
    eps = jnp.float32(1e-16)

    pltpu.sync_copy(sm_hbm, smv)
    smax = smv[...]

    zeros16 = jnp.zeros((16,), jnp.float32)

    def _z1(i, _):
        zb[pl.ds(i * 16, 16)] = zeros16
        return 0
    lax.fori_loop(0, DEN_PER_TILE // 16, _z1, 0)

    def _z2(i, _):
        for k2 in range(8):
            rowb[0, i, pl.ds(k2 * 16, 16)] = zeros16
        return 0
    lax.fori_loop(0, 128, _z2, 0)

    acc_base = sid * ROWS_PER_TILE
    for kk in range(4):
        pltpu.sync_copy(rowb.at[0], acc_sh.at[pl.ds(acc_base + kk * 128, 128)])
    pltpu.sync_copy(rowb.at[0, pl.ds(0, ROWS_PER_TILE - 512)],
                    acc_sh.at[pl.ds(acc_base + 512, ROWS_PER_TILE - 512)])
    pltpu.sync_copy(zb, den_sh.at[pl.ds(sid * DEN_PER_TILE, DEN_PER_TILE)])
    plsc.subcore_barrier()

    def _p1(ch, _):
        pltpu.sync_copy(gsrc.at[c, sid, pl.ds(ch * NGC, NGC)], srcb)
        pltpu.sync_copy(gdst.at[c, sid, pl.ds(ch * NGC, NGC)], dstb)
        for j in range(NGC):
            pltpu.async_copy(s_hbm.at[srcb.at[j]], sbuf.at[j], semA).wait()
            pltpu.async_copy(t_hbm.at[dstb.at[j]], tbuf.at[j], semA).wait()
        for j in range(NGC):
            for k2 in range(8):
                sv = sbuf[j, pl.ds(k2 * 16, 16)]
                tv = tbuf[j, pl.ds(k2 * 16, 16)]
                z = sv + tv
                e = jnp.where(z > 0, z, z * neg)
                z2 = smax + tv
                mv = jnp.where(z2 > 0, z2, z2 * neg)
                wb[j, pl.ds(k2 * 16, 16)] = jnp.exp(e - mv)
        for j in range(NGC):
            pltpu.sync_copy(wb.at[j], den_sh.at[dstb.at[j]], add=True)
        return 0
    lax.fori_loop(0, CH_GAT, _p1, 0)
    plsc.subcore_barrier()

    def _row_pipeline(table_hbm):
        @pl.loop(0, NGC)
        def _rloop(j):
            pltpu.async_copy(
                table_hbm.at[srcb.at[j]], rowb.at[0], semR[0]).wait()

            def _scale(jj, _):
                av = wb[j, pl.ds(jj * 16, 16)]
                for r in range(16):
                    a = av[r]
                    i = jj * 16 + r
                    for k2 in range(8):
                        rowb[0, i, pl.ds(k2 * 16, 16)] = (
                            rowb[0, i, pl.ds(k2 * 16, 16)] * a)
                return 0
            lax.fori_loop(0, 8, _scale, 0)
            pltpu.sync_copy(rowb.at[0], acc_sh.at[dstb.at[j]], add=True)

    def _p2(ch, _):
        pltpu.sync_copy(gsrc.at[c, sid, pl.ds(ch * NGC, NGC)], srcb)
        pltpu.sync_copy(gdst.at[c, sid, pl.ds(ch * NGC, NGC)], dstb)
        for j in range(NGC):
            pltpu.async_copy(s_hbm.at[srcb.at[j]], sbuf.at[j], semA).wait()
            pltpu.async_copy(t_hbm.at[dstb.at[j]], tbuf.at[j], semA).wait()
            pltpu.async_copy(den_sh.at[dstb.at[j]], denb.at[j], semA).wait()
        for j in range(NGC):
            for k2 in range(8):
                sv = sbuf[j, pl.ds(k2 * 16, 16)]
                tv = tbuf[j, pl.ds(k2 * 16, 16)]
                z = sv + tv
                e = jnp.where(z > 0, z, z * neg)
                z2 = smax + tv
                mv = jnp.where(z2 > 0, z2, z2 * neg)
                w = jnp.exp(e - mv)
                dv = denb[j, pl.ds(k2 * 16, 16)]
                wb[j, pl.ds(k2 * 16, 16)] = w / (dv + eps)
        _row_pipeline(h_hbm)
        return 0
    lax.fori_loop(0, CH_GAT, _p2, 0)

    def _p3(ch, _):
        pltpu.sync_copy(psrc.at[wid, pl.ds(ch * NGC, NGC)], srcb)
        pltpu.sync_copy(pdst.at[wid, pl.ds(ch * NGC, NGC)], dstb)
        pltpu.sync_copy(pvals.at[wid, pl.ds(ch * NGC, NGC)], wb)
        _row_pipeline(hp_hbm)
        return 0
    lax.fori_loop(0, CH_P, _p3, 0)
    plsc.subcore_barrier()

    for kk in range(4):
        pltpu.sync_copy(acc_sh.at[pl.ds(acc_base + kk * 128, 128)],
                        out_hbm.at[c, pl.ds(acc_base + kk * 128, 128)])
    pltpu.sync_copy(acc_sh.at[pl.ds(acc_base + 512, ROWS_PER_TILE - 512)],
                    out_hbm.at[c, pl.ds(acc_base + 512, ROWS_PER_TILE - 512)])


def _prep_edges(idx, per_tile, ng, ntile):
    src = idx[0].reshape(ntile, per_tile)
    dst = idx[1].reshape(ntile, per_tile)
    pad = ng * 128 - per_tile
    src = jnp.pad(src, ((0, 0), (0, pad)))
    dst = jnp.pad(dst, ((0, 0), (0, pad)), constant_values=PAD_DST)
    return src.reshape(ntile, ng, 128), dst.reshape(ntile, ng, 128)


def kernel(features, l_u_indices, l_d_indices, p_indices, p_values,
           W_p, b_p, W_d, b_d, a_src_d, a_dst_d):
    A = jnp.zeros((D, D), jnp.float32)
    A = A.at[:, 0].set(a_src_d).at[:, 1].set(a_dst_d)
    h, hp, st, smx = _dense(features, W_d, b_d.reshape(1, D), W_p,
                            b_p.reshape(1, D), A)
    s = st[:, 0]
    t = jnp.pad(st[:, 1], (0, T_PAD - N))
    sm16 = smx[0, :16]

    su, du = _prep_edges(l_u_indices, GAT_PER_TILE, NG_GAT, NT)
    sd, dd = _prep_edges(l_d_indices, GAT_PER_TILE, NG_GAT, NT)
    gsrc = jnp.stack([su, sd])
    gdst = jnp.stack([du, dd])
    psrc, pdst = _prep_edges(p_indices[::-1], P_PER_TILE, NG_P, NT * NSC)
    pv = jnp.pad(p_values.reshape(NT * NSC, P_PER_TILE),
                 ((0, 0), (0, NG_P * 128 - P_PER_TILE)))
    pv = pv.reshape(NT * NSC, NG_P, 128)

    acc = _sc_aggregate(gsrc, gdst, psrc, pdst, pv, h, hp, s, t, sm16)
    return acc[0, :N] + acc[1, :N]


# speedup vs baseline: 6.9563x; 1.0386x over previous
import functools

import jax
import jax.numpy as jnp
from jax import lax
from jax.experimental import pallas as pl
from jax.experimental.pallas import tpu as pltpu
from jax.experimental.pallas import tpu_sc as plsc

N = 10000
D = 128
E = 320000
NT = 16
NSC = 2
NGC = 8
GAT_PER_TILE = E // NT
NG_GAT = 160
CH_GAT = NG_GAT // NGC
P_PER_TILE = E // (NT * NSC)
NG_P = 80
CH_P = NG_P // NGC
PAD_DST = N
T_PAD = N + 16
ACC_ROWS = 10112
DEN_PER_TILE = 640
DEN_WORDS = NT * DEN_PER_TILE
ROWS_PER_TILE = ACC_ROWS // NT


def _dense_body(x_ref, wd_ref, bd_ref, wp_ref, bp_ref, a_ref,
                h_ref, hp_ref, st_ref, sm_ref, smem_ref):
    i = pl.program_id(0)
    x = x_ref[...]
    h = jnp.dot(x, wd_ref[...], preferred_element_type=jnp.float32) + bd_ref[...]
    h_ref[...] = h
    hp_ref[...] = jnp.dot(x, wp_ref[...], preferred_element_type=jnp.float32) + bp_ref[...]
    st = jnp.dot(h, a_ref[...], preferred_element_type=jnp.float32)
    st_ref[...] = st

    @pl.when(i == 0)
    def _():
        smem_ref[0] = jnp.float32(-3.0e38)

    blk_max = jnp.max(st[:, 0])
    smem_ref[0] = jnp.maximum(smem_ref[0], blk_max)

    @pl.when(i == pl.num_programs(0) - 1)
    def _():
        sm_ref[...] = jnp.full((8, 128), smem_ref[0], jnp.float32)


def _dense(x, W_d, b_d, W_p, b_p, A):
    blk = 1000
    grid = N // blk
    return pl.pallas_call(
        _dense_body,
        grid=(grid,),
        in_specs=[
            pl.BlockSpec((blk, D), lambda i: (i, 0)),
            pl.BlockSpec((D, D), lambda i: (0, 0)),
            pl.BlockSpec((1, D), lambda i: (0, 0)),
            pl.BlockSpec((D, D), lambda i: (0, 0)),
            pl.BlockSpec((1, D), lambda i: (0, 0)),
            pl.BlockSpec((D, D), lambda i: (0, 0)),
        ],
        out_specs=[
            pl.BlockSpec((blk, D), lambda i: (i, 0)),
            pl.BlockSpec((blk, D), lambda i: (i, 0)),
            pl.BlockSpec((blk, D), lambda i: (i, 0)),
            pl.BlockSpec((8, 128), lambda i: (0, 0)),
        ],
        out_shape=[
            jax.ShapeDtypeStruct((N, D), jnp.float32),
            jax.ShapeDtypeStruct((N, D), jnp.float32),
            jax.ShapeDtypeStruct((N, D), jnp.float32),
            jax.ShapeDtypeStruct((8, 128), jnp.float32),
        ],
        scratch_shapes=[pltpu.SMEM((1,), jnp.float32)],
    )(x, W_d, b_d, W_p, b_p, A)


_mesh = plsc.VectorSubcoreMesh(core_axis_name="c", subcore_axis_name="s",
                               num_cores=NSC, num_subcores=NT)


@functools.partial(
    pl.kernel,
    out_type=jax.ShapeDtypeStruct((NSC, ACC_ROWS, D), jnp.float32),
    mesh=_mesh,
    compiler_params=pltpu.CompilerParams(needs_layout_passes=False),
    scratch_types=[
        pltpu.VMEM((NGC, 128), jnp.int32),
        pltpu.VMEM((NGC, 128), jnp.int32),
        pltpu.VMEM((NGC, 128), jnp.float32),
        pltpu.VMEM((NGC, 128), jnp.float32),
        pltpu.VMEM((NGC, 128), jnp.float32),
        pltpu.VMEM((NGC, 128), jnp.float32),
        pltpu.VMEM((2, 128, D), jnp.float32),
        pltpu.VMEM((DEN_PER_TILE,), jnp.float32),
        pltpu.VMEM((16,), jnp.float32),
        pltpu.VMEM_SHARED((ACC_ROWS, D), jnp.float32),
        pltpu.VMEM_SHARED((DEN_WORDS,), jnp.float32),
        pltpu.SemaphoreType.DMA,
        pltpu.SemaphoreType.DMA,
        pltpu.SemaphoreType.DMA,
    ],
)
def _sc_aggregate(gsrc, gdst, psrc, pdst, pvals, h_hbm, hp_hbm, s_hbm, t_hbm,
                  sm_hbm, out_hbm, srcb, dstb, wb, sbuf, tbuf, denb,
                  rowb, zb, smv, acc_sh, den_sh, semA, semR0, semR1):
    c = lax.axis_index("c")
    sid = lax.axis_index("s")
    wid = c * NT + sid
    semR = (semR0, semR1)
    neg = jnp.float32(0.2)
    eps = jnp.float32(1e-16)

    pltpu.sync_copy(sm_hbm, smv)
    smax = smv[...]

    zeros16 = jnp.zeros((16,), jnp.float32)

    def _z1(i, _):
        zb[pl.ds(i * 16, 16)] = zeros16
        return 0
    lax.fori_loop(0, DEN_PER_TILE // 16, _z1, 0)

    def _z2(i, _):
        for k2 in range(8):
            rowb[0, i, pl.ds(k2 * 16, 16)] = zeros16
        return 0
    lax.fori_loop(0, 128, _z2, 0)

    acc_base = sid * ROWS_PER_TILE
    for kk in range(4):
        pltpu.sync_copy(rowb.at[0], acc_sh.at[pl.ds(acc_base + kk * 128, 128)])
    pltpu.sync_copy(rowb.at[0, pl.ds(0, ROWS_PER_TILE - 512)],
                    acc_sh.at[pl.ds(acc_base + 512, ROWS_PER_TILE - 512)])
    pltpu.sync_copy(zb, den_sh.at[pl.ds(sid * DEN_PER_TILE, DEN_PER_TILE)])
    plsc.subcore_barrier()

    def _p1(ch, _):
        pltpu.sync_copy(gsrc.at[c, sid, pl.ds(ch * NGC, NGC)], srcb)
        pltpu.sync_copy(gdst.at[c, sid, pl.ds(ch * NGC, NGC)], dstb)
        for j in range(NGC):
            pltpu.async_copy(s_hbm.at[srcb.at[j]], sbuf.at[j], semA).wait()
            pltpu.async_copy(t_hbm.at[dstb.at[j]], tbuf.at[j], semA).wait()
        for j in range(NGC):
            for k2 in range(8):
                sv = sbuf[j, pl.ds(k2 * 16, 16)]
                tv = tbuf[j, pl.ds(k2 * 16, 16)]
                z = sv + tv
                e = jnp.where(z > 0, z, z * neg)
                z2 = smax + tv
                mv = jnp.where(z2 > 0, z2, z2 * neg)
                wb[j, pl.ds(k2 * 16, 16)] = jnp.exp(e - mv)
        for j in range(NGC):
            pltpu.sync_copy(wb.at[j], den_sh.at[dstb.at[j]], add=True)
        return 0
    lax.fori_loop(0, CH_GAT, _p1, 0)
    plsc.subcore_barrier()

    def _scale_group(j, b):
        def _scale(jj, _):
            av = wb[j, pl.ds(jj * 16, 16)]
            for r in range(16):
                a = av[r]
                i = jj * 16 + r
                for k2 in range(8):
                    rowb[b, i, pl.ds(k2 * 16, 16)] = (
                        rowb[b, i, pl.ds(k2 * 16, 16)] * a)
            return 0
        lax.fori_loop(0, 8, _scale, 0)

    def _row_pipeline(table_hbm):
        @pl.loop(0, NGC, step=2)
        def _rloop(g):
            d0 = pltpu.async_copy(table_hbm.at[srcb.at[g]], rowb.at[0],
                                  semR[0])
            d1 = pltpu.async_copy(table_hbm.at[srcb.at[g + 1]], rowb.at[1],
                                  semR[1])
            d0.wait()
            _scale_group(g, 0)
            pltpu.sync_copy(rowb.at[0], acc_sh.at[dstb.at[g]], add=True)
            d1.wait()
            _scale_group(g + 1, 1)
            pltpu.sync_copy(rowb.at[1], acc_sh.at[dstb.at[g + 1]], add=True)

    def _p2(ch, _):
        pltpu.sync_copy(gsrc.at[c, sid, pl.ds(ch * NGC, NGC)], srcb)
        pltpu.sync_copy(gdst.at[c, sid, pl.ds(ch * NGC, NGC)], dstb)
        for j in range(NGC):
            pltpu.async_copy(s_hbm.at[srcb.at[j]], sbuf.at[j], semA).wait()
            pltpu.async_copy(t_hbm.at[dstb.at[j]], tbuf.at[j], semA).wait()
            pltpu.async_copy(den_sh.at[dstb.at[j]], denb.at[j], semA).wait()
        for j in range(NGC):
            for k2 in range(8):
                sv = sbuf[j, pl.ds(k2 * 16, 16)]
                tv = tbuf[j, pl.ds(k2 * 16, 16)]
                z = sv + tv
                e = jnp.where(z > 0, z, z * neg)
                z2 = smax + tv
                mv = jnp.where(z2 > 0, z2, z2 * neg)
                w = jnp.exp(e - mv)
                dv = denb[j, pl.ds(k2 * 16, 16)]
                wb[j, pl.ds(k2 * 16, 16)] = w / (dv + eps)
        _row_pipeline(h_hbm)
        return 0
    lax.fori_loop(0, CH_GAT, _p2, 0)

    def _p3(ch, _):
        pltpu.sync_copy(psrc.at[wid, pl.ds(ch * NGC, NGC)], srcb)
        pltpu.sync_copy(pdst.at[wid, pl.ds(ch * NGC, NGC)], dstb)
        pltpu.sync_copy(pvals.at[wid, pl.ds(ch * NGC, NGC)], wb)
        _row_pipeline(hp_hbm)
        return 0
    lax.fori_loop(0, CH_P, _p3, 0)
    plsc.subcore_barrier()

    for kk in range(4):
        pltpu.sync_copy(acc_sh.at[pl.ds(acc_base + kk * 128, 128)],
                        out_hbm.at[c, pl.ds(acc_base + kk * 128, 128)])
    pltpu.sync_copy(acc_sh.at[pl.ds(acc_base + 512, ROWS_PER_TILE - 512)],
                    out_hbm.at[c, pl.ds(acc_base + 512, ROWS_PER_TILE - 512)])


def _prep_edges(idx, per_tile, ng, ntile):
    src = idx[0].reshape(ntile, per_tile)
    dst = idx[1].reshape(ntile, per_tile)
    pad = ng * 128 - per_tile
    src = jnp.pad(src, ((0, 0), (0, pad)))
    dst = jnp.pad(dst, ((0, 0), (0, pad)), constant_values=PAD_DST)
    return src.reshape(ntile, ng, 128), dst.reshape(ntile, ng, 128)


def kernel(features, l_u_indices, l_d_indices, p_indices, p_values,
           W_p, b_p, W_d, b_d, a_src_d, a_dst_d):
    A = jnp.zeros((D, D), jnp.float32)
    A = A.at[:, 0].set(a_src_d).at[:, 1].set(a_dst_d)
    h, hp, st, smx = _dense(features, W_d, b_d.reshape(1, D), W_p,
                            b_p.reshape(1, D), A)
    s = st[:, 0]
    t = jnp.pad(st[:, 1], (0, T_PAD - N))
    sm16 = smx[0, :16]

    su, du = _prep_edges(l_u_indices, GAT_PER_TILE, NG_GAT, NT)
    sd, dd = _prep_edges(l_d_indices, GAT_PER_TILE, NG_GAT, NT)
    gsrc = jnp.stack([su, sd])
    gdst = jnp.stack([du, dd])
    psrc, pdst = _prep_edges(p_indices[::-1], P_PER_TILE, NG_P, NT * NSC)
    pv = jnp.pad(p_values.reshape(NT * NSC, P_PER_TILE),
                 ((0, 0), (0, NG_P * 128 - P_PER_TILE)))
    pv = pv.reshape(NT * NSC, NG_P, 128)

    acc = _sc_aggregate(gsrc, gdst, psrc, pdst, pv, h, hp, s, t, sm16)
    return acc[0, :N] + acc[1, :N]


# speedup vs baseline: 8.0623x; 1.1590x over previous
import functools

import jax
import jax.numpy as jnp
from jax import lax
from jax.experimental import pallas as pl
from jax.experimental.pallas import tpu as pltpu
from jax.experimental.pallas import tpu_sc as plsc

N = 10000
D = 128
E = 320000
NT = 16
NSC = 2
NGC = 8
GAT_PER_TILE = E // NT
NG_GAT = 160
CH_GAT = NG_GAT // NGC
P_PER_TILE = E // (NT * NSC)
NG_P = 80
CH_P = NG_P // NGC
PAD_DST = N
T_PAD = N + 16
ACC_ROWS = 10112
DEN_PER_TILE = 640
DEN_WORDS = NT * DEN_PER_TILE
ROWS_PER_TILE = ACC_ROWS // NT


def _dense_body(x_ref, wd_ref, bd_ref, wp_ref, bp_ref, a_ref,
                h_ref, hp_ref, st_ref, sm_ref, smem_ref):
    i = pl.program_id(0)
    x = x_ref[...]
    h = jnp.dot(x, wd_ref[...], preferred_element_type=jnp.float32) + bd_ref[...]
    h_ref[...] = h
    hp_ref[...] = jnp.dot(x, wp_ref[...], preferred_element_type=jnp.float32) + bp_ref[...]
    st = jnp.dot(h, a_ref[...], preferred_element_type=jnp.float32)
    st_ref[...] = st

    @pl.when(i == 0)
    def _():
        smem_ref[0] = jnp.float32(-3.0e38)

    blk_max = jnp.max(st[:, 0])
    smem_ref[0] = jnp.maximum(smem_ref[0], blk_max)

    @pl.when(i == pl.num_programs(0) - 1)
    def _():
        sm_ref[...] = jnp.full((8, 128), smem_ref[0], jnp.float32)


def _dense(x, W_d, b_d, W_p, b_p, A):
    blk = 1000
    grid = N // blk
    return pl.pallas_call(
        _dense_body,
        grid=(grid,),
        in_specs=[
            pl.BlockSpec((blk, D), lambda i: (i, 0)),
            pl.BlockSpec((D, D), lambda i: (0, 0)),
            pl.BlockSpec((1, D), lambda i: (0, 0)),
            pl.BlockSpec((D, D), lambda i: (0, 0)),
            pl.BlockSpec((1, D), lambda i: (0, 0)),
            pl.BlockSpec((D, D), lambda i: (0, 0)),
        ],
        out_specs=[
            pl.BlockSpec((blk, D), lambda i: (i, 0)),
            pl.BlockSpec((blk, D), lambda i: (i, 0)),
            pl.BlockSpec((blk, D), lambda i: (i, 0)),
            pl.BlockSpec((8, 128), lambda i: (0, 0)),
        ],
        out_shape=[
            jax.ShapeDtypeStruct((N, D), jnp.float32),
            jax.ShapeDtypeStruct((N, D), jnp.float32),
            jax.ShapeDtypeStruct((N, D), jnp.float32),
            jax.ShapeDtypeStruct((8, 128), jnp.float32),
        ],
        scratch_shapes=[pltpu.SMEM((1,), jnp.float32)],
    )(x, W_d, b_d, W_p, b_p, A)


_mesh = plsc.VectorSubcoreMesh(core_axis_name="c", subcore_axis_name="s",
                               num_cores=NSC, num_subcores=NT)


@functools.partial(
    pl.kernel,
    out_type=jax.ShapeDtypeStruct((NSC, ACC_ROWS, D), jnp.float32),
    mesh=_mesh,
    compiler_params=pltpu.CompilerParams(needs_layout_passes=False),
    scratch_types=[
        pltpu.VMEM((NGC, 128), jnp.int32),
        pltpu.VMEM((NGC, 128), jnp.int32),
        pltpu.VMEM((NGC, 128), jnp.float32),
        pltpu.VMEM((NGC, 128), jnp.float32),
        pltpu.VMEM((NGC, 128), jnp.float32),
        pltpu.VMEM((NGC, 128), jnp.float32),
        pltpu.VMEM((2, 128, D), jnp.float32),
        pltpu.VMEM((DEN_PER_TILE,), jnp.float32),
        pltpu.VMEM((16,), jnp.float32),
        pltpu.VMEM_SHARED((ACC_ROWS, D), jnp.float32),
        pltpu.VMEM_SHARED((DEN_WORDS,), jnp.float32),
        pltpu.SemaphoreType.DMA,
        pltpu.SemaphoreType.DMA,
        pltpu.SemaphoreType.DMA,
        pltpu.SemaphoreType.DMA,
        pltpu.SemaphoreType.DMA,
        pltpu.SemaphoreType.DMA,
    ],
)
def _sc_aggregate(gsrc, gdst, psrc, pdst, pvals, h_hbm, hp_hbm, s_hbm, t_hbm,
                  sm_hbm, out_hbm, srcb, dstb, wb, sbuf, tbuf, denb,
                  rowb, zb, smv, acc_sh, den_sh, semA, semB, semR0, semR1,
                  semS0, semS1):
    c = lax.axis_index("c")
    sid = lax.axis_index("s")
    wid = c * NT + sid
    semR = (semR0, semR1)
    neg = jnp.float32(0.2)
    eps = jnp.float32(1e-16)

    pltpu.sync_copy(sm_hbm, smv)
    smax = smv[...]

    zeros16 = jnp.zeros((16,), jnp.float32)

    def _z1(i, _):
        zb[pl.ds(i * 16, 16)] = zeros16
        return 0
    lax.fori_loop(0, DEN_PER_TILE // 16, _z1, 0)

    def _z2(i, _):
        for k2 in range(8):
            rowb[0, i, pl.ds(k2 * 16, 16)] = zeros16
        return 0
    lax.fori_loop(0, 128, _z2, 0)

    acc_base = sid * ROWS_PER_TILE
    for kk in range(4):
        pltpu.sync_copy(rowb.at[0], acc_sh.at[pl.ds(acc_base + kk * 128, 128)])
    pltpu.sync_copy(rowb.at[0, pl.ds(0, ROWS_PER_TILE - 512)],
                    acc_sh.at[pl.ds(acc_base + 512, ROWS_PER_TILE - 512)])
    pltpu.sync_copy(zb, den_sh.at[pl.ds(sid * DEN_PER_TILE, DEN_PER_TILE)])
    plsc.subcore_barrier()

    def _p1(ch, _):
        pltpu.sync_copy(gsrc.at[c, sid, pl.ds(ch * NGC, NGC)], srcb)
        pltpu.sync_copy(gdst.at[c, sid, pl.ds(ch * NGC, NGC)], dstb)
        for j in range(NGC):
            ds_ = pltpu.async_copy(s_hbm.at[srcb.at[j]], sbuf.at[j], semA)
            dt_ = pltpu.async_copy(t_hbm.at[dstb.at[j]], tbuf.at[j], semB)
            ds_.wait()
            dt_.wait()
        for j in range(NGC):
            for k2 in range(8):
                sv = sbuf[j, pl.ds(k2 * 16, 16)]
                tv = tbuf[j, pl.ds(k2 * 16, 16)]
                z = sv + tv
                e = jnp.where(z > 0, z, z * neg)
                z2 = smax + tv
                mv = jnp.where(z2 > 0, z2, z2 * neg)
                wb[j, pl.ds(k2 * 16, 16)] = jnp.exp(e - mv)
        for j in range(NGC):
            pltpu.sync_copy(wb.at[j], den_sh.at[dstb.at[j]], add=True)
        return 0
    lax.fori_loop(0, CH_GAT, _p1, 0)
    plsc.subcore_barrier()

    def _scale_group(j, b):
        def _scale(jj, _):
            av = wb[j, pl.ds(jj * 16, 16)]
            for r in range(16):
                a = av[r]
                i = jj * 16 + r
                for k2 in range(8):
                    rowb[b, i, pl.ds(k2 * 16, 16)] = (
                        rowb[b, i, pl.ds(k2 * 16, 16)] * a)
            return 0
        lax.fori_loop(0, 8, _scale, 0)

    def _row_pipeline(table_hbm):
        @pl.loop(0, NGC, step=2)
        def _rloop(g):
            d0 = pltpu.async_copy(table_hbm.at[srcb.at[g]], rowb.at[0],
                                  semR[0])
            d1 = pltpu.async_copy(table_hbm.at[srcb.at[g + 1]], rowb.at[1],
                                  semR[1])
            d0.wait()
            _scale_group(g, 0)
            s0 = pltpu.async_copy(rowb.at[0], acc_sh.at[dstb.at[g]], semS0,
                                  add=True)
            d1.wait()
            _scale_group(g + 1, 1)
            s1 = pltpu.async_copy(rowb.at[1], acc_sh.at[dstb.at[g + 1]], semS1,
                                  add=True)
            s0.wait()
            s1.wait()

    def _p2(ch, _):
        pltpu.sync_copy(gsrc.at[c, sid, pl.ds(ch * NGC, NGC)], srcb)
        pltpu.sync_copy(gdst.at[c, sid, pl.ds(ch * NGC, NGC)], dstb)
        for j in range(NGC):
            ds_ = pltpu.async_copy(s_hbm.at[srcb.at[j]], sbuf.at[j], semA)
            dt_ = pltpu.async_copy(t_hbm.at[dstb.at[j]], tbuf.at[j], semB)
            ds_.wait()
            dd_ = pltpu.async_copy(den_sh.at[dstb.at[j]], denb.at[j], semA)
            dt_.wait()
            dd_.wait()
        for j in range(NGC):
            for k2 in range(8):
                sv = sbuf[j, pl.ds(k2 * 16, 16)]
                tv = tbuf[j, pl.ds(k2 * 16, 16)]
                z = sv + tv
                e = jnp.where(z > 0, z, z * neg)
                z2 = smax + tv
                mv = jnp.where(z2 > 0, z2, z2 * neg)
                w = jnp.exp(e - mv)
                dv = denb[j, pl.ds(k2 * 16, 16)]
                wb[j, pl.ds(k2 * 16, 16)] = w / (dv + eps)
        _row_pipeline(h_hbm)
        return 0
    lax.fori_loop(0, CH_GAT, _p2, 0)

    def _p3(ch, _):
        pltpu.sync_copy(psrc.at[wid, pl.ds(ch * NGC, NGC)], srcb)
        pltpu.sync_copy(pdst.at[wid, pl.ds(ch * NGC, NGC)], dstb)
        pltpu.sync_copy(pvals.at[wid, pl.ds(ch * NGC, NGC)], wb)
        _row_pipeline(hp_hbm)
        return 0
    lax.fori_loop(0, CH_P, _p3, 0)
    plsc.subcore_barrier()

    for kk in range(4):
        pltpu.sync_copy(acc_sh.at[pl.ds(acc_base + kk * 128, 128)],
                        out_hbm.at[c, pl.ds(acc_base + kk * 128, 128)])
    pltpu.sync_copy(acc_sh.at[pl.ds(acc_base + 512, ROWS_PER_TILE - 512)],
                    out_hbm.at[c, pl.ds(acc_base + 512, ROWS_PER_TILE - 512)])


def _prep_edges(idx, per_tile, ng, ntile):
    src = idx[0].reshape(ntile, per_tile)
    dst = idx[1].reshape(ntile, per_tile)
    pad = ng * 128 - per_tile
    src = jnp.pad(src, ((0, 0), (0, pad)))
    dst = jnp.pad(dst, ((0, 0), (0, pad)), constant_values=PAD_DST)
    return src.reshape(ntile, ng, 128), dst.reshape(ntile, ng, 128)


def kernel(features, l_u_indices, l_d_indices, p_indices, p_values,
           W_p, b_p, W_d, b_d, a_src_d, a_dst_d):
    A = jnp.zeros((D, D), jnp.float32)
    A = A.at[:, 0].set(a_src_d).at[:, 1].set(a_dst_d)
    h, hp, st, smx = _dense(features, W_d, b_d.reshape(1, D), W_p,
                            b_p.reshape(1, D), A)
    s = st[:, 0]
    t = jnp.pad(st[:, 1], (0, T_PAD - N))
    sm16 = smx[0, :16]

    su, du = _prep_edges(l_u_indices, GAT_PER_TILE, NG_GAT, NT)
    sd, dd = _prep_edges(l_d_indices, GAT_PER_TILE, NG_GAT, NT)
    gsrc = jnp.stack([su, sd])
    gdst = jnp.stack([du, dd])
    psrc, pdst = _prep_edges(p_indices[::-1], P_PER_TILE, NG_P, NT * NSC)
    pv = jnp.pad(p_values.reshape(NT * NSC, P_PER_TILE),
                 ((0, 0), (0, NG_P * 128 - P_PER_TILE)))
    pv = pv.reshape(NT * NSC, NG_P, 128)

    acc = _sc_aggregate(gsrc, gdst, psrc, pdst, pv, h, hp, s, t, sm16)
    return acc[0, :N] + acc[1, :N]
